# baseline (speedup 1.0000x reference)
import jax
import jax.numpy as jnp
from jax import lax
from jax.experimental import pallas as pl

_FIELD = 38462
_F = 26
_E = 16


def _tile_kernel(x_ref, w_ref, o_ref):
    w = w_ref[:, :_E]
    tiled = jnp.concatenate([w] * _F, axis=1)
    col_f = lax.broadcasted_iota(jnp.int32, (_F, _F * _E), 1) // _E
    row_f = lax.broadcasted_iota(jnp.int32, (_F, _F * _E), 0)
    m = jnp.where(col_f == row_f, tiled, 0.0)
    xf = x_ref[...].astype(jnp.float32)
    o_ref[...] = jnp.dot(xf, m, preferred_element_type=jnp.float32)


@jax.jit
def kernel(x, weight):
    B = x.shape[0]
    bt = 2048
    w2 = weight.reshape(_F, _FIELD * _E)
    out = pl.pallas_call(
        _tile_kernel,
        grid=(B // bt,),
        in_specs=[
            pl.BlockSpec((bt, _F), lambda i: (i, 0)),
            pl.BlockSpec((_F, 128), lambda i: (0, 0)),
        ],
        out_specs=pl.BlockSpec((bt, _F * _E), lambda i: (i, 0)),
        out_shape=jax.ShapeDtypeStruct((B, _F * _E), jnp.float32),
    )(x, w2)
    return out


# speedup vs baseline: 1.8372x; 1.8372x over previous
import jax
import jax.numpy as jnp
from jax import lax
from jax.experimental import pallas as pl
from jax.experimental.pallas import tpu as pltpu

_FIELD = 38462
_F = 26
_E = 16


def _tile_kernel(x_ref, w_hbm, o_ref, w_vmem, sem):
    @pl.when(pl.program_id(0) == 0)
    def _load_rows():
        for f in range(_F):
            pltpu.make_async_copy(
                w_hbm.at[pl.ds(f * _FIELD, 1), :],
                w_vmem.at[pl.ds(f, 1), :],
                sem,
            ).start()
        for f in range(_F):
            pltpu.make_async_copy(
                w_hbm.at[pl.ds(f * _FIELD, 1), :],
                w_vmem.at[pl.ds(f, 1), :],
                sem,
            ).wait()

    w = w_vmem[...]
    tiled = jnp.concatenate([w] * _F, axis=1)
    col_f = lax.broadcasted_iota(jnp.int32, (_F, _F * _E), 1) // _E
    row_f = lax.broadcasted_iota(jnp.int32, (_F, _F * _E), 0)
    m = jnp.where(col_f == row_f, tiled, 0.0)
    xf = x_ref[...].astype(jnp.float32)
    o_ref[...] = jnp.dot(xf, m, preferred_element_type=jnp.float32)


@jax.jit
def kernel(x, weight):
    B = x.shape[0]
    bt = 2048
    out = pl.pallas_call(
        _tile_kernel,
        grid=(B // bt,),
        in_specs=[
            pl.BlockSpec((bt, _F), lambda i: (i, 0)),
            pl.BlockSpec(memory_space=pl.ANY),
        ],
        out_specs=pl.BlockSpec((bt, _F * _E), lambda i: (i, 0)),
        out_shape=jax.ShapeDtypeStruct((B, _F * _E), jnp.float32),
        scratch_shapes=[
            pltpu.VMEM((_F, _E), jnp.float32),
            pltpu.SemaphoreType.DMA,
        ],
    )(x, weight)
    return out
